# baseline (speedup 1.0000x reference)
import functools

import jax
import jax.numpy as jnp
from jax import lax
from jax.experimental import pallas as pl
from jax.experimental.pallas import tpu as pltpu
from jax.experimental.pallas import tpu_sc as plsc

NC = 2
NS = 16
NW = NC * NS
CH = 128

F32 = jnp.float32


def _sc_mesh():
    return plsc.VectorSubcoreMesh(core_axis_name="c", subcore_axis_name="s")


def _deg_call(dst2d, zeros1d, ones_ch, npad, epad):
    rpt = npad // NS
    epw = epad // NW
    nch = epw // CH
    K = 10

    def body(dst_hbm, zeros_hbm, ones_hbm, out_hbm, idx_v, ones_v, wb_v,
             deg_sh, sem):
        cid = lax.axis_index("c")
        sid = lax.axis_index("s")
        wid = cid * NS + sid
        pltpu.sync_copy(zeros_hbm, wb_v)
        pltpu.sync_copy(wb_v, deg_sh.at[pl.ds(sid * rpt, rpt)])
        pltpu.sync_copy(ones_hbm, ones_v)
        pltpu.sync_copy(dst_hbm.at[pl.ds(wid * nch, nch)], idx_v)
        plsc.subcore_barrier()

        def grp(h, carry):
            descs = []
            for b in range(2 * K):
                descs.append(pltpu.async_copy(
                    ones_v, deg_sh.at[idx_v.at[h * 2 * K + b]], sem, add=True))
            for dsc in descs:
                dsc.wait()
            return carry

        lax.fori_loop(0, nch // (2 * K), grp, 0)
        plsc.subcore_barrier()
        pltpu.sync_copy(deg_sh.at[pl.ds(sid * rpt, rpt)], wb_v)
        off = pl.multiple_of(cid * npad + sid * rpt, 8)
        pltpu.sync_copy(wb_v, out_hbm.at[pl.ds(off, rpt)])

    f = pl.kernel(
        body,
        out_type=jax.ShapeDtypeStruct((NC * npad,), F32),
        mesh=_sc_mesh(),
        scratch_types=[
            pltpu.VMEM((nch, CH), jnp.int32),
            pltpu.VMEM((CH,), F32),
            pltpu.VMEM((rpt,), F32),
            pltpu.VMEM_SHARED((npad,), F32),
            pltpu.SemaphoreType.DMA,
        ],
    )
    return f(dst2d, zeros1d, ones_ch)


def _agg_call(src2d, dst2d, table, zeros2d, npad, epad, d):
    rpt = npad // NS
    epw = epad // NW
    nch = epw // CH
    K = 10

    def body(src_hbm, dst_hbm, tbl_hbm, zeros_hbm, out_hbm,
             sidx_v, didx_v, rows_a, rows_b, wb_v, acc_sh,
             gsem_a, gsem_b, ssem_a, ssem_b):
        cid = lax.axis_index("c")
        sid = lax.axis_index("s")
        wid = cid * NS + sid
        pltpu.sync_copy(zeros_hbm, wb_v)
        pltpu.sync_copy(wb_v, acc_sh.at[pl.ds(sid * rpt, rpt)])
        pltpu.sync_copy(src_hbm.at[pl.ds(wid * nch, nch)], sidx_v)
        pltpu.sync_copy(dst_hbm.at[pl.ds(wid * nch, nch)], didx_v)
        plsc.subcore_barrier()

        def grp(h, carry):
            g0 = h * 2 * K
            g1 = g0 + K
            ga = [pltpu.async_copy(tbl_hbm.at[sidx_v.at[g0 + b]],
                                   rows_a.at[b], gsem_a) for b in range(K)]
            gb = [pltpu.async_copy(tbl_hbm.at[sidx_v.at[g1 + b]],
                                   rows_b.at[b], gsem_b) for b in range(K)]
            for dsc in ga:
                dsc.wait()
            sa = [pltpu.async_copy(rows_a.at[b], acc_sh.at[didx_v.at[g0 + b]],
                                   ssem_a, add=True) for b in range(K)]
            for dsc in gb:
                dsc.wait()
            sb = [pltpu.async_copy(rows_b.at[b], acc_sh.at[didx_v.at[g1 + b]],
                                   ssem_b, add=True) for b in range(K)]
            for dsc in sa:
                dsc.wait()
            for dsc in sb:
                dsc.wait()
            return carry

        lax.fori_loop(0, nch // (2 * K), grp, 0)
        plsc.subcore_barrier()
        pltpu.sync_copy(acc_sh.at[pl.ds(sid * rpt, rpt)], wb_v)
        off = pl.multiple_of(sid * rpt, 8)
        pltpu.sync_copy(wb_v, out_hbm.at[cid, pl.ds(off, rpt)])

    f = pl.kernel(
        body,
        out_type=jax.ShapeDtypeStruct((NC, npad, d), F32),
        mesh=_sc_mesh(),
        compiler_params=pltpu.CompilerParams(use_tc_tiling_on_sc=False),
        scratch_types=[
            pltpu.VMEM((nch, CH), jnp.int32),
            pltpu.VMEM((nch, CH), jnp.int32),
            pltpu.VMEM((K, CH, d), F32),
            pltpu.VMEM((K, CH, d), F32),
            pltpu.VMEM((rpt, d), F32),
            pltpu.VMEM_SHARED((npad, d), F32),
            pltpu.SemaphoreType.DMA,
            pltpu.SemaphoreType.DMA,
            pltpu.SemaphoreType.DMA,
            pltpu.SemaphoreType.DMA,
        ],
    )
    return f(src2d, dst2d, table, zeros2d)


def _tc1_call(x_p, W1, d0, d1, npad, bm, d_in, d):
    def body(x_ref, w_ref, d0_ref, d1_ref, h_ref, dinv_ref):
        deg = 1.0 + d0_ref[...] + d1_ref[...]
        dinv = lax.rsqrt(deg)
        h = jnp.dot(x_ref[...], w_ref[...], preferred_element_type=F32)
        h_ref[...] = h * dinv
        dinv_ref[...] = dinv

    grid = (npad // bm,)
    return pl.pallas_call(
        body,
        grid=grid,
        in_specs=[
            pl.BlockSpec((bm, d_in), lambda i: (i, 0)),
            pl.BlockSpec((d_in, d), lambda i: (0, 0)),
            pl.BlockSpec((bm, 1), lambda i: (i, 0)),
            pl.BlockSpec((bm, 1), lambda i: (i, 0)),
        ],
        out_specs=[
            pl.BlockSpec((bm, d), lambda i: (i, 0)),
            pl.BlockSpec((bm, 1), lambda i: (i, 0)),
        ],
        out_shape=[
            jax.ShapeDtypeStruct((npad, d), F32),
            jax.ShapeDtypeStruct((npad, 1), F32),
        ],
    )(x_p, W1, d0, d1)


def _tc2_call(h1p, p0, p1, dinv, b1, npad, bm, d):
    def body(h_ref, p0_ref, p1_ref, dinv_ref, b_ref, z_ref):
        acc = h_ref[...] + p0_ref[...] + p1_ref[...]
        z = jnp.maximum(acc * dinv_ref[...] + b_ref[...], 0.0)
        z_ref[...] = z * dinv_ref[...]

    grid = (npad // bm,)
    return pl.pallas_call(
        body,
        grid=grid,
        in_specs=[
            pl.BlockSpec((bm, d), lambda i: (i, 0)),
            pl.BlockSpec((bm, d), lambda i: (i, 0)),
            pl.BlockSpec((bm, d), lambda i: (i, 0)),
            pl.BlockSpec((bm, 1), lambda i: (i, 0)),
            pl.BlockSpec((1, d), lambda i: (0, 0)),
        ],
        out_specs=pl.BlockSpec((bm, d), lambda i: (i, 0)),
        out_shape=jax.ShapeDtypeStruct((npad, d), F32),
    )(h1p, p0, p1, dinv, b1)


def _tc3_call(z1p, q0, q1, dinv, W2, b2, npad, bm, d, d_out):
    def body(z_ref, q0_ref, q1_ref, dinv_ref, w_ref, b_ref, o_ref):
        agg = (z_ref[...] + q0_ref[...] + q1_ref[...]) * dinv_ref[...]
        o_ref[...] = (
            jnp.dot(agg, w_ref[...], preferred_element_type=F32) + b_ref[...]
        )

    grid = (npad // bm,)
    return pl.pallas_call(
        body,
        grid=grid,
        in_specs=[
            pl.BlockSpec((bm, d), lambda i: (i, 0)),
            pl.BlockSpec((bm, d), lambda i: (i, 0)),
            pl.BlockSpec((bm, d), lambda i: (i, 0)),
            pl.BlockSpec((bm, 1), lambda i: (i, 0)),
            pl.BlockSpec((d, d_out), lambda i: (0, 0)),
            pl.BlockSpec((1, d_out), lambda i: (0, 0)),
        ],
        out_specs=pl.BlockSpec((bm, d_out), lambda i: (i, 0)),
        out_shape=jax.ShapeDtypeStruct((npad, d_out), F32),
    )(z1p, q0, q1, dinv, W2, b2)


def kernel(x, edge_index, W1, b1, W2, b2):
    N, d_in = x.shape
    d = W1.shape[1]
    d_out = W2.shape[1]
    E = edge_index.shape[1]

    rpt = -(-N // (NS * 8)) * 8
    npad = rpt * NS
    epw = -(-E // (NW * CH * 20)) * CH * 20
    epad = epw * NW
    nch = epw // CH
    bm = 2 * rpt

    src = edge_index[0]
    dst = edge_index[1]
    pad_e = epad - E
    src_p = jnp.concatenate([src, jnp.zeros((pad_e,), edge_index.dtype)])
    dst_p = jnp.concatenate([dst, jnp.full((pad_e,), N, edge_index.dtype)])
    src2d = src_p.reshape(NW * nch, CH)
    dst2d = dst_p.reshape(NW * nch, CH)
    x_p = jnp.pad(x, ((0, npad - N), (0, 0)))

    zeros1d = jnp.zeros((rpt,), F32)
    zeros2d = jnp.zeros((rpt, d), F32)
    ones_ch = jnp.ones((CH,), F32)

    deg2 = _deg_call(dst2d, zeros1d, ones_ch, npad, epad).reshape(NC, npad)
    d0 = deg2[0][:, None]
    d1 = deg2[1][:, None]

    h1p, dinv = _tc1_call(x_p, W1, d0, d1, npad, bm, d_in, d)
    p = _agg_call(src2d, dst2d, h1p, zeros2d, npad, epad, d)
    z1p = _tc2_call(h1p, p[0], p[1], dinv, b1[None, :], npad, bm, d)
    q = _agg_call(src2d, dst2d, z1p, zeros2d, npad, epad, d)
    outp = _tc3_call(z1p, q[0], q[1], dinv, W2, b2[None, :], npad, bm, d, d_out)
    return outp[:N]


# speedup vs baseline: 1.1608x; 1.1608x over previous
import functools

import jax
import jax.numpy as jnp
from jax import lax
from jax.experimental import pallas as pl
from jax.experimental.pallas import tpu as pltpu
from jax.experimental.pallas import tpu_sc as plsc

NC = 2
NS = 16
NW = NC * NS
CH = 128

F32 = jnp.float32


def _sc_mesh():
    return plsc.VectorSubcoreMesh(core_axis_name="c", subcore_axis_name="s")


def _deg_call(dst2d, zeros1d, ones_ch, npad, epad):
    rpt = npad // NS
    epw = epad // NW
    nch = epw // CH
    K = 8

    def body(dst_hbm, zeros_hbm, ones_hbm, out_hbm, idx_v, ones_v, wb_v,
             deg_sh, sem):
        cid = lax.axis_index("c")
        sid = lax.axis_index("s")
        wid = cid * NS + sid
        pltpu.sync_copy(zeros_hbm, wb_v)
        pltpu.sync_copy(wb_v, deg_sh.at[pl.ds(sid * rpt, rpt)])
        pltpu.sync_copy(ones_hbm, ones_v)
        pltpu.sync_copy(dst_hbm.at[pl.ds(wid * nch, nch)], idx_v)
        plsc.subcore_barrier()

        def grp(h, carry):
            descs = []
            for b in range(2 * K):
                descs.append(pltpu.async_copy(
                    ones_v, deg_sh.at[idx_v.at[h * 2 * K + b]], sem, add=True))
            for dsc in descs:
                dsc.wait()
            return carry

        lax.fori_loop(0, nch // (2 * K), grp, 0)
        plsc.subcore_barrier()
        pltpu.sync_copy(deg_sh.at[pl.ds(sid * rpt, rpt)], wb_v)
        off = pl.multiple_of(cid * npad + sid * rpt, 8)
        pltpu.sync_copy(wb_v, out_hbm.at[pl.ds(off, rpt)])

    f = pl.kernel(
        body,
        out_type=jax.ShapeDtypeStruct((NC * npad,), F32),
        mesh=_sc_mesh(),
        scratch_types=[
            pltpu.VMEM((nch, CH), jnp.int32),
            pltpu.VMEM((CH,), F32),
            pltpu.VMEM((rpt,), F32),
            pltpu.VMEM_SHARED((npad,), F32),
            pltpu.SemaphoreType.DMA,
        ],
    )
    return f(dst2d, zeros1d, ones_ch)


def _agg_call(src2d, dst2d, table, zeros2d, npad, epad, d):
    rpt = npad // NS
    epw = epad // NW
    nch = epw // CH
    K = 8

    def body(src_hbm, dst_hbm, tbl_hbm, zeros_hbm, out_hbm,
             sidx_v, didx_v, rows_a, rows_b, wb_v, acc_sh,
             gsem_a, gsem_b, ssem_a, ssem_b):
        cid = lax.axis_index("c")
        sid = lax.axis_index("s")
        wid = cid * NS + sid
        pltpu.sync_copy(zeros_hbm, wb_v)
        pltpu.sync_copy(wb_v, acc_sh.at[pl.ds(sid * rpt, rpt)])
        pltpu.sync_copy(src_hbm.at[pl.ds(wid * nch, nch)], sidx_v)
        pltpu.sync_copy(dst_hbm.at[pl.ds(wid * nch, nch)], didx_v)
        plsc.subcore_barrier()

        def grp(h, carry):
            g0 = h * 2 * K
            g1 = g0 + K
            ga = [pltpu.async_copy(tbl_hbm.at[sidx_v.at[g0 + b]],
                                   rows_a.at[b], gsem_a) for b in range(K)]
            gb = [pltpu.async_copy(tbl_hbm.at[sidx_v.at[g1 + b]],
                                   rows_b.at[b], gsem_b) for b in range(K)]
            for dsc in ga:
                dsc.wait()
            sa = [pltpu.async_copy(rows_a.at[b], acc_sh.at[didx_v.at[g0 + b]],
                                   ssem_a, add=True) for b in range(K)]
            for dsc in gb:
                dsc.wait()
            sb = [pltpu.async_copy(rows_b.at[b], acc_sh.at[didx_v.at[g1 + b]],
                                   ssem_b, add=True) for b in range(K)]
            for dsc in sa:
                dsc.wait()
            for dsc in sb:
                dsc.wait()
            return carry

        lax.fori_loop(0, nch // (2 * K), grp, 0)
        plsc.subcore_barrier()
        pltpu.sync_copy(acc_sh.at[pl.ds(sid * rpt, rpt)], wb_v)
        off = pl.multiple_of(sid * rpt, 8)
        pltpu.sync_copy(wb_v, out_hbm.at[cid, pl.ds(off, rpt)])

    f = pl.kernel(
        body,
        out_type=jax.ShapeDtypeStruct((NC, npad, d), F32),
        mesh=_sc_mesh(),
        compiler_params=pltpu.CompilerParams(use_tc_tiling_on_sc=False),
        scratch_types=[
            pltpu.VMEM((nch, CH), jnp.int32),
            pltpu.VMEM((nch, CH), jnp.int32),
            pltpu.VMEM((K, CH, d), F32),
            pltpu.VMEM((K, CH, d), F32),
            pltpu.VMEM((rpt, d), F32),
            pltpu.VMEM_SHARED((npad, d), F32),
            pltpu.SemaphoreType.DMA,
            pltpu.SemaphoreType.DMA,
            pltpu.SemaphoreType.DMA,
            pltpu.SemaphoreType.DMA,
        ],
    )
    return f(src2d, dst2d, table, zeros2d)


def _tc1_call(x_p, W1, d0, d1, npad, bm, d_in, d):
    def body(x_ref, w_ref, d0_ref, d1_ref, h_ref, dinv_ref):
        deg = 1.0 + d0_ref[...] + d1_ref[...]
        dinv = lax.rsqrt(deg)
        h = jnp.dot(x_ref[...], w_ref[...], preferred_element_type=F32)
        h_ref[...] = h * dinv
        dinv_ref[...] = dinv

    grid = (npad // bm,)
    return pl.pallas_call(
        body,
        grid=grid,
        in_specs=[
            pl.BlockSpec((bm, d_in), lambda i: (i, 0)),
            pl.BlockSpec((d_in, d), lambda i: (0, 0)),
            pl.BlockSpec((bm, 1), lambda i: (i, 0)),
            pl.BlockSpec((bm, 1), lambda i: (i, 0)),
        ],
        out_specs=[
            pl.BlockSpec((bm, d), lambda i: (i, 0)),
            pl.BlockSpec((bm, 1), lambda i: (i, 0)),
        ],
        out_shape=[
            jax.ShapeDtypeStruct((npad, d), F32),
            jax.ShapeDtypeStruct((npad, 1), F32),
        ],
    )(x_p, W1, d0, d1)


def _tc2_call(h1p, p0, p1, dinv, b1, npad, bm, d):
    def body(h_ref, p0_ref, p1_ref, dinv_ref, b_ref, z_ref):
        acc = h_ref[...] + p0_ref[...] + p1_ref[...]
        z = jnp.maximum(acc * dinv_ref[...] + b_ref[...], 0.0)
        z_ref[...] = z * dinv_ref[...]

    grid = (npad // bm,)
    return pl.pallas_call(
        body,
        grid=grid,
        in_specs=[
            pl.BlockSpec((bm, d), lambda i: (i, 0)),
            pl.BlockSpec((bm, d), lambda i: (i, 0)),
            pl.BlockSpec((bm, d), lambda i: (i, 0)),
            pl.BlockSpec((bm, 1), lambda i: (i, 0)),
            pl.BlockSpec((1, d), lambda i: (0, 0)),
        ],
        out_specs=pl.BlockSpec((bm, d), lambda i: (i, 0)),
        out_shape=jax.ShapeDtypeStruct((npad, d), F32),
    )(h1p, p0, p1, dinv, b1)


def _tc3_call(z1p, q0, q1, dinv, W2, b2, npad, bm, d, d_out):
    def body(z_ref, q0_ref, q1_ref, dinv_ref, w_ref, b_ref, o_ref):
        agg = (z_ref[...] + q0_ref[...] + q1_ref[...]) * dinv_ref[...]
        o_ref[...] = (
            jnp.dot(agg, w_ref[...], preferred_element_type=F32) + b_ref[...]
        )

    grid = (npad // bm,)
    return pl.pallas_call(
        body,
        grid=grid,
        in_specs=[
            pl.BlockSpec((bm, d), lambda i: (i, 0)),
            pl.BlockSpec((bm, d), lambda i: (i, 0)),
            pl.BlockSpec((bm, d), lambda i: (i, 0)),
            pl.BlockSpec((bm, 1), lambda i: (i, 0)),
            pl.BlockSpec((d, d_out), lambda i: (0, 0)),
            pl.BlockSpec((1, d_out), lambda i: (0, 0)),
        ],
        out_specs=pl.BlockSpec((bm, d_out), lambda i: (i, 0)),
        out_shape=jax.ShapeDtypeStruct((npad, d_out), F32),
    )(z1p, q0, q1, dinv, W2, b2)


def kernel(x, edge_index, W1, b1, W2, b2):
    N, d_in = x.shape
    d = W1.shape[1]
    d_out = W2.shape[1]
    E = edge_index.shape[1]

    rpt = -(-N // (NS * 8)) * 8
    npad = rpt * NS
    epw = -(-E // (NW * CH * 16)) * CH * 16
    epad = epw * NW
    nch = epw // CH
    bm = 2 * rpt

    src = edge_index[0]
    dst = edge_index[1]
    ppw = epw - E // NW
    pad_src = jnp.zeros((NW, ppw), edge_index.dtype)
    pad_dst = jnp.broadcast_to(
        N + (jnp.arange(ppw, dtype=edge_index.dtype) % (npad - N)), (NW, ppw))
    src2d = jnp.concatenate(
        [src.reshape(NW, E // NW), pad_src], axis=1).reshape(NW * nch, CH)
    dst2d = jnp.concatenate(
        [dst.reshape(NW, E // NW), pad_dst], axis=1).reshape(NW * nch, CH)
    x_p = jnp.pad(x, ((0, npad - N), (0, 0)))

    zeros1d = jnp.zeros((rpt,), F32)
    zeros2d = jnp.zeros((rpt, d), F32)
    ones_ch = jnp.ones((CH,), F32)

    deg2 = _deg_call(dst2d, zeros1d, ones_ch, npad, epad).reshape(NC, npad)
    d0 = deg2[0][:, None]
    d1 = deg2[1][:, None]

    h1p, dinv = _tc1_call(x_p, W1, d0, d1, npad, bm, d_in, d)
    p = _agg_call(src2d, dst2d, h1p, zeros2d, npad, epad, d)
    z1p = _tc2_call(h1p, p[0], p[1], dinv, b1[None, :], npad, bm, d)
    q = _agg_call(src2d, dst2d, z1p, zeros2d, npad, epad, d)
    outp = _tc3_call(z1p, q[0], q[1], dinv, W2, b2[None, :], npad, bm, d, d_out)
    return outp[:N]


# speedup vs baseline: 1.7689x; 1.5239x over previous
import functools

import jax
import jax.numpy as jnp
from jax import lax
from jax.experimental import pallas as pl
from jax.experimental.pallas import tpu as pltpu
from jax.experimental.pallas import tpu_sc as plsc

NC = 2
NS = 16
NW = NC * NS
CH = 100

F32 = jnp.float32


def _sc_mesh():
    return plsc.VectorSubcoreMesh(core_axis_name="c", subcore_axis_name="s")


def _deg_call(dst2d, zeros1d, ones_ch, npad, epad):
    rpt = npad // NS
    epw = epad // NW
    nch = epw // CH
    K = 10

    def body(dst_hbm, zeros_hbm, ones_hbm, out_hbm, idx_v, ones_v, wb_v,
             deg_sh, sem):
        cid = lax.axis_index("c")
        sid = lax.axis_index("s")
        wid = cid * NS + sid
        pltpu.sync_copy(zeros_hbm, wb_v)
        pltpu.sync_copy(wb_v, deg_sh.at[pl.ds(sid * rpt, rpt)])
        pltpu.sync_copy(ones_hbm, ones_v)
        pltpu.sync_copy(dst_hbm.at[pl.ds(wid * nch, nch)], idx_v)
        plsc.subcore_barrier()

        def grp(h, carry):
            descs = []
            for b in range(2 * K):
                descs.append(pltpu.async_copy(
                    ones_v, deg_sh.at[idx_v.at[h * 2 * K + b]], sem, add=True))
            for dsc in descs:
                dsc.wait()
            return carry

        lax.fori_loop(0, nch // (2 * K), grp, 0)
        plsc.subcore_barrier()
        pltpu.sync_copy(deg_sh.at[pl.ds(sid * rpt, rpt)], wb_v)
        off = pl.multiple_of(cid * npad + sid * rpt, 8)
        pltpu.sync_copy(wb_v, out_hbm.at[pl.ds(off, rpt)])

    f = pl.kernel(
        body,
        out_type=jax.ShapeDtypeStruct((NC * npad,), F32),
        mesh=_sc_mesh(),
        compiler_params=pltpu.CompilerParams(use_tc_tiling_on_sc=False),
        scratch_types=[
            pltpu.VMEM((nch, CH), jnp.int32),
            pltpu.VMEM((CH,), F32),
            pltpu.VMEM((rpt,), F32),
            pltpu.VMEM_SHARED((npad,), F32),
            pltpu.SemaphoreType.DMA,
        ],
    )
    return f(dst2d, zeros1d, ones_ch)


def _agg_call(src2d, dst2d, table, zeros2d, npad, epad, d):
    rpt = npad // NS
    epw = epad // NW
    nch = epw // CH
    K = 10

    def body(src_hbm, dst_hbm, tbl_hbm, zeros_hbm, out_hbm,
             sidx_v, didx_v, rows_a, rows_b, wb_v, acc_sh,
             gsem_a, gsem_b, ssem_a, ssem_b):
        cid = lax.axis_index("c")
        sid = lax.axis_index("s")
        wid = cid * NS + sid
        pltpu.sync_copy(zeros_hbm, wb_v)
        pltpu.sync_copy(wb_v, acc_sh.at[pl.ds(sid * rpt, rpt)])
        pltpu.sync_copy(src_hbm.at[pl.ds(wid * nch, nch)], sidx_v)
        pltpu.sync_copy(dst_hbm.at[pl.ds(wid * nch, nch)], didx_v)
        plsc.subcore_barrier()

        def grp(h, carry):
            g0 = h * 2 * K
            g1 = g0 + K
            ga = [pltpu.async_copy(tbl_hbm.at[sidx_v.at[g0 + b]],
                                   rows_a.at[b], gsem_a) for b in range(K)]
            gb = [pltpu.async_copy(tbl_hbm.at[sidx_v.at[g1 + b]],
                                   rows_b.at[b], gsem_b) for b in range(K)]
            for dsc in ga:
                dsc.wait()
            sa = [pltpu.async_copy(rows_a.at[b], acc_sh.at[didx_v.at[g0 + b]],
                                   ssem_a, add=True) for b in range(K)]
            for dsc in gb:
                dsc.wait()
            sb = [pltpu.async_copy(rows_b.at[b], acc_sh.at[didx_v.at[g1 + b]],
                                   ssem_b, add=True) for b in range(K)]
            for dsc in sa:
                dsc.wait()
            for dsc in sb:
                dsc.wait()
            return carry

        lax.fori_loop(0, nch // (2 * K), grp, 0)
        plsc.subcore_barrier()
        pltpu.sync_copy(acc_sh.at[pl.ds(sid * rpt, rpt)], wb_v)
        off = pl.multiple_of(sid * rpt, 8)
        pltpu.sync_copy(wb_v, out_hbm.at[cid, pl.ds(off, rpt)])

    f = pl.kernel(
        body,
        out_type=jax.ShapeDtypeStruct((NC, npad, d), F32),
        mesh=_sc_mesh(),
        compiler_params=pltpu.CompilerParams(use_tc_tiling_on_sc=False),
        scratch_types=[
            pltpu.VMEM((nch, CH), jnp.int32),
            pltpu.VMEM((nch, CH), jnp.int32),
            pltpu.VMEM((K, CH, d), F32),
            pltpu.VMEM((K, CH, d), F32),
            pltpu.VMEM((rpt, d), F32),
            pltpu.VMEM_SHARED((npad, d), F32),
            pltpu.SemaphoreType.DMA,
            pltpu.SemaphoreType.DMA,
            pltpu.SemaphoreType.DMA,
            pltpu.SemaphoreType.DMA,
        ],
    )
    return f(src2d, dst2d, table, zeros2d)


def _tc1_call(x, W1, d0, d1, npad, d_in, d):
    N = x.shape[0]

    def body(x_ref, w_ref, d0_ref, d1_ref, h_ref, dinv_ref):
        deg = 1.0 + d0_ref[...] + d1_ref[...]
        dinv = lax.rsqrt(deg)
        h = jnp.dot(x_ref[...], w_ref[...], preferred_element_type=F32)
        hp = h * dinv[:N]
        h_ref[...] = jnp.concatenate(
            [hp, jnp.zeros((npad - N, d), F32)], axis=0)
        dinv_ref[...] = dinv

    return pl.pallas_call(
        body,
        out_shape=[
            jax.ShapeDtypeStruct((npad, d), F32),
            jax.ShapeDtypeStruct((npad, 1), F32),
        ],
    )(x, W1, d0, d1)


def _tc2_call(h1p, pp, dinv, b1, npad, d):
    def body(h_ref, p_ref, dinv_ref, b_ref, z_ref):
        acc = h_ref[...] + jnp.sum(p_ref[...], axis=0)
        dv = dinv_ref[...]
        z = jnp.maximum(acc * dv + b_ref[...], 0.0)
        z_ref[...] = z * dv

    return pl.pallas_call(
        body,
        out_shape=jax.ShapeDtypeStruct((npad, d), F32),
    )(h1p, pp, dinv, b1)


def _tc3_call(z1p, qq, dinv, W2, b2t, N, npad, d, d_out):
    def body(z_ref, q_ref, dinv_ref, w_ref, b_ref, o_ref):
        agg = (z_ref[...] + jnp.sum(q_ref[...], axis=0)) * dinv_ref[...]
        o_ref[...] = (
            jnp.dot(agg[:N], w_ref[...], preferred_element_type=F32)
            + b_ref[...]
        )

    return pl.pallas_call(
        body,
        out_shape=jax.ShapeDtypeStruct((N, d_out), F32),
    )(z1p, qq, dinv, W2, b2t)


def kernel(x, edge_index, W1, b1, W2, b2):
    N, d_in = x.shape
    d = W1.shape[1]
    d_out = W2.shape[1]
    E = edge_index.shape[1]

    rpt = -(-N // (NS * 8)) * 8
    npad = rpt * NS
    epw = E // NW
    epad = E
    nch = epw // CH

    src2d = edge_index[0].reshape(NW * nch, CH)
    dst2d = edge_index[1].reshape(NW * nch, CH)

    zeros1d = jnp.zeros((rpt,), F32)
    zeros2d = jnp.zeros((rpt, d), F32)
    ones_ch = jnp.ones((CH,), F32)

    deg2 = _deg_call(dst2d, zeros1d, ones_ch, npad, epad)
    d0 = deg2[:npad, None]
    d1 = deg2[npad:, None]

    h1p, dinv = _tc1_call(x, W1, d0, d1, npad, d_in, d)
    p = _agg_call(src2d, dst2d, h1p, zeros2d, npad, epad, d)
    z1p = _tc2_call(h1p, p, dinv, b1[None, :], npad, d)
    q = _agg_call(src2d, dst2d, z1p, zeros2d, npad, epad, d)
    return _tc3_call(z1p, q, dinv, W2, b2[None, :], N, npad, d, d_out)


# speedup vs baseline: 1.8701x; 1.0572x over previous
import functools

import jax
import jax.numpy as jnp
from jax import lax
from jax.experimental import pallas as pl
from jax.experimental.pallas import tpu as pltpu
from jax.experimental.pallas import tpu_sc as plsc

NC = 2
NS = 16
NW = NC * NS
CH = 100

F32 = jnp.float32


def _sc_mesh():
    return plsc.VectorSubcoreMesh(core_axis_name="c", subcore_axis_name="s")


def _deg_call(er, zeros1d, ones_ch, npad, epad):
    rpt = npad // NS
    epw = epad // NW
    nch = epw // CH
    K = 10

    def body(er_hbm, zeros_hbm, ones_hbm, out_hbm, idx_v, ones_v, wb_v,
             deg_sh, sem):
        cid = lax.axis_index("c")
        sid = lax.axis_index("s")
        wid = cid * NS + sid
        pltpu.sync_copy(zeros_hbm, wb_v)
        pltpu.sync_copy(wb_v, deg_sh.at[pl.ds(sid * rpt, rpt)])
        pltpu.sync_copy(ones_hbm, ones_v)
        pltpu.sync_copy(er_hbm.at[1].at[pl.ds(wid * nch, nch)], idx_v)
        plsc.subcore_barrier()

        def grp(h, carry):
            descs = []
            for b in range(2 * K):
                descs.append(pltpu.async_copy(
                    ones_v, deg_sh.at[idx_v.at[h * 2 * K + b]], sem, add=True))
            for dsc in descs:
                dsc.wait()
            return carry

        lax.fori_loop(0, nch // (2 * K), grp, 0)
        plsc.subcore_barrier()
        pltpu.sync_copy(deg_sh.at[pl.ds(sid * rpt, rpt)], wb_v)
        off = pl.multiple_of(cid * npad + sid * rpt, 8)
        pltpu.sync_copy(wb_v, out_hbm.at[pl.ds(off, rpt)])

    f = pl.kernel(
        body,
        out_type=jax.ShapeDtypeStruct((NC * npad,), F32),
        mesh=_sc_mesh(),
        compiler_params=pltpu.CompilerParams(use_tc_tiling_on_sc=False),
        scratch_types=[
            pltpu.VMEM((nch, CH), jnp.int32),
            pltpu.VMEM((CH,), F32),
            pltpu.VMEM((rpt,), F32),
            pltpu.VMEM_SHARED((npad,), F32),
            pltpu.SemaphoreType.DMA,
        ],
    )
    return f(er, zeros1d, ones_ch)


def _agg_call(er, table, zeros2d, npad, epad, d):
    rpt = npad // NS
    epw = epad // NW
    nch = epw // CH
    K = 10

    def body(er_hbm, tbl_hbm, zeros_hbm, out_hbm,
             sidx_v, didx_v, rows_a, rows_b, wb_v, acc_sh,
             gsem_a, gsem_b, ssem_a, ssem_b):
        cid = lax.axis_index("c")
        sid = lax.axis_index("s")
        wid = cid * NS + sid
        pltpu.sync_copy(zeros_hbm, wb_v)
        pltpu.sync_copy(wb_v, acc_sh.at[pl.ds(sid * rpt, rpt)])
        pltpu.sync_copy(er_hbm.at[0].at[pl.ds(wid * nch, nch)], sidx_v)
        pltpu.sync_copy(er_hbm.at[1].at[pl.ds(wid * nch, nch)], didx_v)
        plsc.subcore_barrier()

        def grp(h, carry):
            g0 = h * 2 * K
            g1 = g0 + K
            ga = [pltpu.async_copy(tbl_hbm.at[sidx_v.at[g0 + b]],
                                   rows_a.at[b], gsem_a) for b in range(K)]
            gb = [pltpu.async_copy(tbl_hbm.at[sidx_v.at[g1 + b]],
                                   rows_b.at[b], gsem_b) for b in range(K)]
            for dsc in ga:
                dsc.wait()
            sa = [pltpu.async_copy(rows_a.at[b], acc_sh.at[didx_v.at[g0 + b]],
                                   ssem_a, add=True) for b in range(K)]
            for dsc in gb:
                dsc.wait()
            sb = [pltpu.async_copy(rows_b.at[b], acc_sh.at[didx_v.at[g1 + b]],
                                   ssem_b, add=True) for b in range(K)]
            for dsc in sa:
                dsc.wait()
            for dsc in sb:
                dsc.wait()
            return carry

        lax.fori_loop(0, nch // (2 * K), grp, 0)
        plsc.subcore_barrier()
        pltpu.sync_copy(acc_sh.at[pl.ds(sid * rpt, rpt)], wb_v)
        off = pl.multiple_of(sid * rpt, 8)
        pltpu.sync_copy(wb_v, out_hbm.at[cid, pl.ds(off, rpt)])

    f = pl.kernel(
        body,
        out_type=jax.ShapeDtypeStruct((NC, npad, d), F32),
        mesh=_sc_mesh(),
        compiler_params=pltpu.CompilerParams(use_tc_tiling_on_sc=False),
        scratch_types=[
            pltpu.VMEM((nch, CH), jnp.int32),
            pltpu.VMEM((nch, CH), jnp.int32),
            pltpu.VMEM((K, CH, d), F32),
            pltpu.VMEM((K, CH, d), F32),
            pltpu.VMEM((rpt, d), F32),
            pltpu.VMEM_SHARED((npad, d), F32),
            pltpu.SemaphoreType.DMA,
            pltpu.SemaphoreType.DMA,
            pltpu.SemaphoreType.DMA,
            pltpu.SemaphoreType.DMA,
        ],
    )
    return f(er, table, zeros2d)


def _tc1_call(x, W1, dsum, npad, d_in, d):
    N = x.shape[0]

    def body(x_ref, w_ref, dsum_ref, h_ref, dinv_ref):
        deg = 1.0 + dsum_ref[...]
        dinv = lax.rsqrt(deg)
        h = jnp.dot(x_ref[...], w_ref[...], preferred_element_type=F32)
        hp = h * dinv[:N]
        h_ref[...] = jnp.concatenate(
            [hp, jnp.zeros((npad - N, d), F32)], axis=0)
        dinv_ref[...] = dinv

    return pl.pallas_call(
        body,
        out_shape=[
            jax.ShapeDtypeStruct((npad, d), F32),
            jax.ShapeDtypeStruct((npad, 1), F32),
        ],
    )(x, W1, dsum)


def _tc2_call(h1p, pp, dinv, b1, npad, d):
    def body(h_ref, p_ref, dinv_ref, b_ref, z_ref):
        acc = h_ref[...] + jnp.sum(p_ref[...], axis=0)
        dv = dinv_ref[...]
        z = jnp.maximum(acc * dv + b_ref[...], 0.0)
        z_ref[...] = z * dv

    return pl.pallas_call(
        body,
        out_shape=jax.ShapeDtypeStruct((npad, d), F32),
    )(h1p, pp, dinv, b1)


def _tc3_call(z1p, qq, dinv, W2, b2t, N, npad, d, d_out):
    def body(z_ref, q_ref, dinv_ref, w_ref, b_ref, o_ref):
        agg = (z_ref[...] + jnp.sum(q_ref[...], axis=0)) * dinv_ref[...]
        o_ref[...] = (
            jnp.dot(agg[:N], w_ref[...], preferred_element_type=F32)
            + b_ref[...]
        )

    return pl.pallas_call(
        body,
        out_shape=jax.ShapeDtypeStruct((N, d_out), F32),
    )(z1p, qq, dinv, W2, b2t)


def kernel(x, edge_index, W1, b1, W2, b2):
    N, d_in = x.shape
    d = W1.shape[1]
    d_out = W2.shape[1]
    E = edge_index.shape[1]

    rpt = -(-N // (NS * 8)) * 8
    npad = rpt * NS
    epw = E // NW
    epad = E
    nch = epw // CH

    er = edge_index.reshape(2, NW * nch, CH)

    zeros1d = jnp.zeros((rpt,), F32)
    zeros2d = jnp.zeros((rpt, d), F32)
    ones_ch = jnp.ones((CH,), F32)

    deg2 = _deg_call(er, zeros1d, ones_ch, npad, epad)
    dsum = (deg2[:npad] + deg2[npad:])[:, None]

    h1p, dinv = _tc1_call(x, W1, dsum, npad, d_in, d)
    p = _agg_call(er, h1p, zeros2d, npad, epad, d)
    z1p = _tc2_call(h1p, p, dinv, b1[None, :], npad, d)
    q = _agg_call(er, z1p, zeros2d, npad, epad, d)
    return _tc3_call(z1p, q, dinv, W2, b2[None, :], N, npad, d, d_out)


# speedup vs baseline: 1.8770x; 1.0037x over previous
import jax
import jax.numpy as jnp
from jax import lax
from jax.experimental import pallas as pl
from jax.experimental.pallas import tpu as pltpu
from jax.experimental.pallas import tpu_sc as plsc

NC = 2
NS = 16
NW = NC * NS
CH = 100

F32 = jnp.float32


def _sc_mesh():
    return plsc.VectorSubcoreMesh(core_axis_name="c", subcore_axis_name="s")


def _deg_call(er, zeros1d, ones_ch, npad, epad):
    rpt = npad // NS
    epw = epad // NW
    nch = epw // CH
    K = 10

    def body(er_hbm, zeros_hbm, ones_hbm, out_hbm, idx_v, ones_v, wb_v,
             deg_sh, sem):
        cid = lax.axis_index("c")
        sid = lax.axis_index("s")
        wid = cid * NS + sid
        pltpu.sync_copy(zeros_hbm, wb_v)
        pltpu.sync_copy(wb_v, deg_sh.at[pl.ds(sid * rpt, rpt)])
        pltpu.sync_copy(ones_hbm, ones_v)
        pltpu.sync_copy(er_hbm.at[1].at[pl.ds(wid * nch, nch)], idx_v)
        plsc.subcore_barrier()

        def grp(h, carry):
            descs = []
            for b in range(2 * K):
                descs.append(pltpu.async_copy(
                    ones_v, deg_sh.at[idx_v.at[h * 2 * K + b]], sem, add=True))
            for dsc in descs:
                dsc.wait()
            return carry

        lax.fori_loop(0, nch // (2 * K), grp, 0)
        plsc.subcore_barrier()
        pltpu.sync_copy(deg_sh.at[pl.ds(sid * rpt, rpt)], wb_v)
        off = pl.multiple_of(cid * npad + sid * rpt, 8)
        pltpu.sync_copy(wb_v, out_hbm.at[pl.ds(off, rpt)])

    f = pl.kernel(
        body,
        out_type=jax.ShapeDtypeStruct((NC * npad,), F32),
        mesh=_sc_mesh(),
        compiler_params=pltpu.CompilerParams(use_tc_tiling_on_sc=False),
        scratch_types=[
            pltpu.VMEM((nch, CH), jnp.int32),
            pltpu.VMEM((CH,), F32),
            pltpu.VMEM((rpt,), F32),
            pltpu.VMEM_SHARED((npad,), F32),
            pltpu.SemaphoreType.DMA,
        ],
    )
    return f(er, zeros1d, ones_ch)


def _agg_call(er, table, zeros2d, npad, epad, d):
    rpt = npad // NS
    epw = epad // NW
    nch = epw // CH
    K = 10

    def body(er_hbm, tbl_hbm, zeros_hbm, out_hbm,
             sidx_v, didx_v, rows_a, rows_b, wb_v, acc_sh,
             gsem_a, gsem_b, ssem_a, ssem_b):
        cid = lax.axis_index("c")
        sid = lax.axis_index("s")
        wid = cid * NS + sid
        pltpu.sync_copy(zeros_hbm, wb_v)
        pltpu.sync_copy(wb_v, acc_sh.at[pl.ds(sid * rpt, rpt)])
        pltpu.sync_copy(er_hbm.at[0].at[pl.ds(wid * nch, nch)], sidx_v)
        pltpu.sync_copy(er_hbm.at[1].at[pl.ds(wid * nch, nch)], didx_v)
        plsc.subcore_barrier()

        def grp(h, carry):
            g0 = h * 2 * K
            g1 = g0 + K
            ga = [pltpu.async_copy(tbl_hbm.at[sidx_v.at[g0 + b]],
                                   rows_a.at[b], gsem_a) for b in range(K)]
            gb = [pltpu.async_copy(tbl_hbm.at[sidx_v.at[g1 + b]],
                                   rows_b.at[b], gsem_b) for b in range(K)]
            for dsc in ga:
                dsc.wait()
            sa = [pltpu.async_copy(rows_a.at[b], acc_sh.at[didx_v.at[g0 + b]],
                                   ssem_a, add=True) for b in range(K)]
            for dsc in gb:
                dsc.wait()
            sb = [pltpu.async_copy(rows_b.at[b], acc_sh.at[didx_v.at[g1 + b]],
                                   ssem_b, add=True) for b in range(K)]
            for dsc in sa:
                dsc.wait()
            for dsc in sb:
                dsc.wait()
            return carry

        lax.fori_loop(0, nch // (2 * K), grp, 0)
        plsc.subcore_barrier()
        pltpu.sync_copy(acc_sh.at[pl.ds(sid * rpt, rpt)], wb_v)
        off = pl.multiple_of(sid * rpt, 8)
        pltpu.sync_copy(wb_v, out_hbm.at[cid, pl.ds(off, rpt)])

    f = pl.kernel(
        body,
        out_type=jax.ShapeDtypeStruct((NC, npad, d), F32),
        mesh=_sc_mesh(),
        compiler_params=pltpu.CompilerParams(use_tc_tiling_on_sc=False),
        scratch_types=[
            pltpu.VMEM((nch, CH), jnp.int32),
            pltpu.VMEM((nch, CH), jnp.int32),
            pltpu.VMEM((K, CH, d), F32),
            pltpu.VMEM((K, CH, d), F32),
            pltpu.VMEM((rpt, d), F32),
            pltpu.VMEM_SHARED((npad, d), F32),
            pltpu.SemaphoreType.DMA,
            pltpu.SemaphoreType.DMA,
            pltpu.SemaphoreType.DMA,
            pltpu.SemaphoreType.DMA,
        ],
    )
    return f(er, table, zeros2d)


def _tc1_call(x, W1, dsum, npad, d_in, d):
    N = x.shape[0]

    def body(x_ref, w_ref, dsum_ref, h_ref, dinv_ref):
        deg = 1.0 + dsum_ref[...]
        dinv = lax.rsqrt(deg)
        h = jnp.dot(x_ref[...], w_ref[...], preferred_element_type=F32)
        hp = h * dinv[:N]
        h_ref[...] = jnp.concatenate(
            [hp, jnp.zeros((npad - N, d), F32)], axis=0)
        dinv_ref[...] = dinv

    return pl.pallas_call(
        body,
        out_shape=[
            jax.ShapeDtypeStruct((npad, d), F32),
            jax.ShapeDtypeStruct((npad, 1), F32),
        ],
    )(x, W1, dsum)


def _tc2_call(h1p, pp, dinv, b1, npad, d):
    def body(h_ref, p_ref, dinv_ref, b_ref, z_ref):
        acc = h_ref[...] + jnp.sum(p_ref[...], axis=0)
        dv = dinv_ref[...]
        z = jnp.maximum(acc * dv + b_ref[...], 0.0)
        z_ref[...] = z * dv

    return pl.pallas_call(
        body,
        out_shape=jax.ShapeDtypeStruct((npad, d), F32),
    )(h1p, pp, dinv, b1)


def _tc3_call(z1p, qq, dinv, W2, b2t, N, npad, d, d_out):
    def body(z_ref, q_ref, dinv_ref, w_ref, b_ref, o_ref):
        agg = (z_ref[...] + jnp.sum(q_ref[...], axis=0)) * dinv_ref[...]
        o_ref[...] = (
            jnp.dot(agg[:N], w_ref[...], preferred_element_type=F32)
            + b_ref[...]
        )

    return pl.pallas_call(
        body,
        out_shape=jax.ShapeDtypeStruct((N, d_out), F32),
    )(z1p, qq, dinv, W2, b2t)


def kernel(x, edge_index, W1, b1, W2, b2):
    N, d_in = x.shape
    d = W1.shape[1]
    d_out = W2.shape[1]
    E = edge_index.shape[1]

    rpt = -(-N // (NS * 8)) * 8
    npad = rpt * NS
    epw = E // NW
    epad = E
    nch = epw // CH

    er = edge_index.reshape(2, NW * nch, CH)

    zeros1d = jnp.zeros((rpt,), F32)
    zeros2d = jnp.zeros((rpt, d), F32)
    ones_ch = jnp.ones((CH,), F32)

    deg2 = _deg_call(er, zeros1d, ones_ch, npad, epad)
    dsum = (deg2[:npad] + deg2[npad:])[:, None]

    h1p, dinv = _tc1_call(x, W1, dsum, npad, d_in, d)
    p = _agg_call(er, h1p, zeros2d, npad, epad, d)
    z1p = _tc2_call(h1p, p, dinv, b1[None, :], npad, d)
    q = _agg_call(er, z1p, zeros2d, npad, epad, d)
    return _tc3_call(z1p, q, dinv, W2, b2[None, :], N, npad, d, d_out)
